# Initial kernel scaffold; baseline (speedup 1.0000x reference)
import jax
import jax.numpy as jnp
from jax.experimental import pallas as pl


def kernel(ids, emb_matrix):
    raise NotImplementedError("write your pallas kernel here")


# speedup vs baseline: 1.0223x; 1.0223x over previous
import functools

import jax
import jax.numpy as jnp
from jax import lax
from jax.experimental import pallas as pl
from jax.experimental.pallas import tpu as pltpu
from jax.experimental.pallas import tpu_sc as plsc

NUM_EMBEDDINGS = 1_000_000
EMB_DIM = 32
BATCH = 16384 * 50
NUM_CORES = 2
NUM_SUBCORES = 16
NUM_WORKERS = NUM_CORES * NUM_SUBCORES
PER_WORKER = BATCH // NUM_WORKERS
CHUNK = 128
NUM_CHUNKS = PER_WORKER // CHUNK

_mesh = plsc.VectorSubcoreMesh(core_axis_name="c", subcore_axis_name="s")


@functools.partial(
    pl.kernel,
    mesh=_mesh,
    out_type=jax.ShapeDtypeStruct((BATCH, EMB_DIM), jnp.float32),
    scratch_types=[
        pltpu.VMEM((NUM_CHUNKS, CHUNK), jnp.int32),
        pltpu.VMEM((CHUNK, EMB_DIM), jnp.float32),
        pltpu.SemaphoreType.DMA,
    ],
    compiler_params=pltpu.CompilerParams(use_tc_tiling_on_sc=False),
)
def _gather_kernel(ids_hbm, table_hbm, out_hbm, idx_v, rows_v, gsem):
    wid = lax.axis_index("s") * NUM_CORES + lax.axis_index("c")
    base = wid * PER_WORKER
    pltpu.sync_copy(ids_hbm.at[wid], idx_v)

    def body(j, carry):
        pltpu.async_copy(table_hbm.at[idx_v.at[j]], rows_v, gsem).wait()
        pltpu.sync_copy(rows_v, out_hbm.at[pl.ds(base + j * CHUNK, CHUNK)])
        return carry

    lax.fori_loop(0, NUM_CHUNKS, body, 0)


@jax.jit
def kernel(ids, emb_matrix):
    ids32 = ids.astype(jnp.int32).reshape(NUM_WORKERS, NUM_CHUNKS, CHUNK)
    out = _gather_kernel(ids32, emb_matrix)
    return out.reshape(*ids.shape, EMB_DIM)


# speedup vs baseline: 1.1091x; 1.0849x over previous
import functools

import jax
import jax.numpy as jnp
from jax import lax
from jax.experimental import pallas as pl
from jax.experimental.pallas import tpu as pltpu
from jax.experimental.pallas import tpu_sc as plsc

NUM_EMBEDDINGS = 1_000_000
EMB_DIM = 32
BATCH = 16384 * 50
NUM_CORES = 2
NUM_SUBCORES = 16
NUM_WORKERS = NUM_CORES * NUM_SUBCORES
PER_WORKER = BATCH // NUM_WORKERS
CHUNK = 128
NUM_CHUNKS = PER_WORKER // CHUNK
NBUF = 4

_mesh = plsc.VectorSubcoreMesh(core_axis_name="c", subcore_axis_name="s")


@functools.partial(
    pl.kernel,
    mesh=_mesh,
    out_type=jax.ShapeDtypeStruct((BATCH, EMB_DIM), jnp.float32),
    scratch_types=[
        pltpu.VMEM((NUM_CHUNKS, CHUNK), jnp.int32),
        pltpu.VMEM((NBUF, CHUNK, EMB_DIM), jnp.float32),
        pltpu.SemaphoreType.DMA((NBUF,)),
        pltpu.SemaphoreType.DMA((NBUF,)),
    ],
    compiler_params=pltpu.CompilerParams(use_tc_tiling_on_sc=False),
)
def _gather_kernel(ids_hbm, table_hbm, out_hbm, idx_v, rows_v, gsem, wsem):
    wid = lax.axis_index("s") * NUM_CORES + lax.axis_index("c")
    base = wid * PER_WORKER
    pltpu.sync_copy(ids_hbm.at[wid], idx_v)

    for j in range(NBUF - 1):
        pltpu.async_copy(table_hbm.at[idx_v.at[j]], rows_v.at[j], gsem.at[j])

    def body(j, carry):
        b = lax.rem(j, NBUF)
        bn = lax.rem(j + NBUF - 1, NBUF)

        @pl.when(j + NBUF - 1 < NUM_CHUNKS)
        def _():
            @pl.when(j >= 1)
            def _():
                pltpu.make_async_copy(
                    rows_v.at[bn], out_hbm.at[pl.ds(base, CHUNK)], wsem.at[bn]
                ).wait()
            pltpu.async_copy(
                table_hbm.at[idx_v.at[j + NBUF - 1]], rows_v.at[bn], gsem.at[bn]
            )

        pltpu.make_async_copy(
            table_hbm.at[idx_v.at[j]], rows_v.at[b], gsem.at[b]
        ).wait()
        pltpu.async_copy(
            rows_v.at[b], out_hbm.at[pl.ds(base + j * CHUNK, CHUNK)], wsem.at[b]
        )
        return carry

    lax.fori_loop(0, NUM_CHUNKS, body, 0)

    for b in range(NBUF):
        pltpu.make_async_copy(
            rows_v.at[b], out_hbm.at[pl.ds(base, CHUNK)], wsem.at[b]
        ).wait()


@jax.jit
def kernel(ids, emb_matrix):
    ids32 = ids.astype(jnp.int32).reshape(NUM_WORKERS, NUM_CHUNKS, CHUNK)
    out = _gather_kernel(ids32, emb_matrix)
    return out.reshape(*ids.shape, EMB_DIM)


# speedup vs baseline: 1.7886x; 1.6126x over previous
import functools

import jax
import jax.numpy as jnp
from jax import lax
from jax.experimental import pallas as pl
from jax.experimental.pallas import tpu as pltpu
from jax.experimental.pallas import tpu_sc as plsc

NUM_EMBEDDINGS = 1_000_000
EMB_DIM = 32
BATCH_B = 16384
SEQ_T = 50
NUM_CORES = 2
NUM_SUBCORES = 16
NUM_WORKERS = NUM_CORES * NUM_SUBCORES
PER_B = BATCH_B // NUM_WORKERS
KROWS = 16
NUM_STEPS = PER_B // KROWS
NBUF = 2

_mesh = plsc.VectorSubcoreMesh(core_axis_name="c", subcore_axis_name="s")


@functools.partial(
    pl.kernel,
    mesh=_mesh,
    out_type=jax.ShapeDtypeStruct((BATCH_B, SEQ_T, EMB_DIM), jnp.float32),
    scratch_types=[
        pltpu.VMEM((PER_B, SEQ_T), jnp.int32),
        pltpu.VMEM((NBUF, KROWS, SEQ_T, EMB_DIM), jnp.float32),
        pltpu.SemaphoreType.DMA((NBUF,)),
        pltpu.SemaphoreType.DMA((NBUF,)),
    ],
    compiler_params=pltpu.CompilerParams(use_tc_tiling_on_sc=False),
)
def _gather_kernel(ids_hbm, table_hbm, out_hbm, idx_v, rows_v, gsem, wsem):
    wid = lax.axis_index("s") * NUM_CORES + lax.axis_index("c")
    base_b = wid * PER_B
    pltpu.sync_copy(ids_hbm.at[pl.ds(base_b, PER_B)], idx_v)

    def body(g, carry):
        p = lax.rem(g, NBUF)

        @pl.when(g >= NBUF)
        def _():
            pltpu.make_async_copy(
                rows_v.at[p], out_hbm.at[pl.ds(base_b, KROWS)], wsem.at[p]
            ).wait()

        for k in range(KROWS):
            pltpu.async_copy(
                table_hbm.at[idx_v.at[g * KROWS + k]],
                rows_v.at[p].at[k],
                gsem.at[p],
            )
        pltpu.make_async_copy(
            out_hbm.at[pl.ds(base_b, KROWS)], rows_v.at[p], gsem.at[p]
        ).wait()
        pltpu.async_copy(
            rows_v.at[p],
            out_hbm.at[pl.ds(base_b + g * KROWS, KROWS)],
            wsem.at[p],
        )
        return carry

    lax.fori_loop(0, NUM_STEPS, body, 0)

    for p in range(NBUF):
        pltpu.make_async_copy(
            rows_v.at[p], out_hbm.at[pl.ds(base_b, KROWS)], wsem.at[p]
        ).wait()


@jax.jit
def kernel(ids, emb_matrix):
    return _gather_kernel(ids.astype(jnp.int32), emb_matrix)
